# baseline (speedup 1.0000x reference)
import functools

import jax
import jax.numpy as jnp
from jax import lax
from jax.experimental import pallas as pl
from jax.experimental.pallas import tpu as pltpu
from jax.experimental.pallas import tpu_sc as plsc

B = 2048
D_IN = 1024
D_SAE = 16384
K_MAX = 131072
NTH = 9

_INTERPRET = False


def _encode_body(x_ref, w_ref, b_ref, a_ref, mx_ref):
    j = pl.program_id(0)
    acc = jax.lax.dot_general(
        x_ref[...], w_ref[...], (((1,), (1,)), ((), ())),
        preferred_element_type=jnp.float32)
    a = jnp.maximum(acc + b_ref[...], 0.0)
    a_ref[...] = a
    m = jnp.max(a)

    @pl.when(j == 0)
    def _():
        mx_ref[...] = jnp.full((1, 1), m, jnp.float32)

    @pl.when(j > 0)
    def _():
        mx_ref[...] = jnp.maximum(mx_ref[...], jnp.full((1, 1), m, jnp.float32))


def _encode(x, W_enc_w, W_enc_b):
    nsteps = 16
    bn = D_SAE // nsteps
    return pl.pallas_call(
        _encode_body,
        grid=(nsteps,),
        in_specs=[
            pl.BlockSpec((B, D_IN), lambda j: (0, 0)),
            pl.BlockSpec((bn, D_IN), lambda j: (j, 0)),
            pl.BlockSpec((1, bn), lambda j: (0, j)),
        ],
        out_specs=[
            pl.BlockSpec((B, bn), lambda j: (0, j)),
            pl.BlockSpec((1, 1), lambda j: (0, 0)),
        ],
        out_shape=[
            jax.ShapeDtypeStruct((B, D_SAE), jnp.float32),
            jax.ShapeDtypeStruct((1, 1), jnp.float32),
        ],
        interpret=_INTERPRET,
    )(x, W_enc_w, W_enc_b.reshape(1, D_SAE))


def _count_body(thr_ref, a_ref, cnt_ref, *, nsteps):
    j = pl.program_id(0)
    bits = jax.lax.bitcast_convert_type(a_ref[...], jnp.int32)

    @pl.when(j == 0)
    def _():
        for i in range(NTH):
            cnt_ref[i] = 0

    for i in range(NTH):
        cnt_ref[i] += jnp.sum((bits >= thr_ref[i]).astype(jnp.int32))


def _count_pass(a, thr_bits, rows, nsteps):
    bm = rows // nsteps
    return pl.pallas_call(
        functools.partial(_count_body, nsteps=nsteps),
        grid=(nsteps,),
        in_specs=[
            pl.BlockSpec(memory_space=pltpu.SMEM),
            pl.BlockSpec((bm, D_SAE), lambda j: (j, 0)),
        ],
        out_specs=pl.BlockSpec(memory_space=pltpu.SMEM),
        out_shape=jax.ShapeDtypeStruct((NTH,), jnp.int32),
        interpret=_INTERPRET,
    )(thr_bits, a)


_NLANE = 16
_NBIN = 4096
_CH = 8192
_NELEM = B * D_SAE


def _sc_hist_body(level, a_ref, par_ref, out_ref, buf0, buf1, hist, fold,
                  shared, pv, idxv, sem0, sem1):
    cid = lax.axis_index("c")
    sid = lax.axis_index("s")
    wid = sid * 2 + cid
    nw = 32
    slab = _NELEM // nw
    nch = slab // _CH
    base = wid * slab

    lane_base = jnp.arange(_NLANE, dtype=jnp.int32) * _NBIN
    ones = jnp.ones((_NLANE,), jnp.int32)
    zeros16 = jnp.zeros((_NLANE,), jnp.int32)
    l16 = jnp.arange(_NLANE, dtype=jnp.int32)

    def zhist(i, c):
        hist[pl.ds(i * _NLANE, _NLANE)] = zeros16
        return c

    lax.fori_loop(0, (_NBIN * _NLANE) // _NLANE, zhist, 0)

    def zfold(i, c):
        fold[i // 8, pl.ds((i % 8) * _NLANE, _NLANE)] = zeros16
        return c

    lax.fori_loop(0, _NBIN // _NLANE, zfold, 0)

    idxv[pl.ds(0, _NLANE)] = l16
    idxv[pl.ds(_NLANE, _NLANE)] = l16 + _NLANE

    @pl.when(sid == 0)
    def _():
        pltpu.sync_copy(fold, shared)

    if level == 2:
        pltpu.sync_copy(par_ref.at[0], pv)
    plsc.subcore_barrier()

    def process(buf):
        bsel = pv[...] if level == 2 else None

        def ib(i, c):
            for u in range(8):
                bits = buf[pl.ds(i * 128 + u * 16, 16)]
                top = lax.shift_right_logical(bits, 19)
                if level == 1:
                    idx = lane_base + top
                    plsc.addupdate_scatter(hist, [idx], ones)
                else:
                    mid = jnp.bitwise_and(
                        lax.shift_right_logical(bits, 7), _NBIN - 1)
                    idx = lane_base + mid
                    plsc.addupdate_scatter(hist, [idx], ones,
                                           mask=top == bsel)
            return c

        lax.fori_loop(0, _CH // 128, ib, 0)

    pltpu.async_copy(a_ref.at[pl.ds(base, _CH)], buf0, sem0)

    def pair(p, c):
        pltpu.async_copy(
            a_ref.at[pl.ds(base + (2 * p + 1) * _CH, _CH)], buf1, sem1)
        pltpu.make_async_copy(a_ref.at[pl.ds(0, _CH)], buf0, sem0).wait()
        process(buf0)

        @pl.when(2 * p + 2 < nch)
        def _():
            pltpu.async_copy(
                a_ref.at[pl.ds(base + (2 * p + 2) * _CH, _CH)], buf0, sem0)

        pltpu.make_async_copy(a_ref.at[pl.ds(0, _CH)], buf1, sem1).wait()
        process(buf1)
        return c

    lax.fori_loop(0, nch // 2, pair, 0)

    def fld(cb, c):
        acc = zeros16
        for l in range(_NLANE):
            acc = acc + hist[pl.ds(cb * _NLANE + l * _NBIN, _NLANE)]
        fold[cb // 8, pl.ds((cb % 8) * _NLANE, _NLANE)] = acc
        return c

    lax.fori_loop(0, _NBIN // _NLANE, fld, 0)

    pltpu.sync_copy(fold, shared.at[idxv], add=True)
    plsc.subcore_barrier()

    @pl.when(sid == 0)
    def _():
        pltpu.sync_copy(shared, out_ref.at[cid])


def _sc_hist(a_flat, params, level):
    mesh = plsc.VectorSubcoreMesh(core_axis_name="c", subcore_axis_name="s")
    return pl.kernel(
        functools.partial(_sc_hist_body, level),
        mesh=mesh,
        compiler_params=pltpu.CompilerParams(needs_layout_passes=False),
        out_type=jax.ShapeDtypeStruct((2, _NBIN // 128, 128), jnp.int32),
        scratch_types=[
            pltpu.VMEM((_CH,), jnp.int32),
            pltpu.VMEM((_CH,), jnp.int32),
            pltpu.VMEM((_NBIN * _NLANE,), jnp.int32),
            pltpu.VMEM((_NBIN // 128, 128), jnp.int32),
            pltpu.VMEM_SHARED((_NBIN // 128, 128), jnp.int32),
            pltpu.VMEM((_NLANE,), jnp.int32),
            pltpu.VMEM((_NBIN // 128,), jnp.int32),
            pltpu.SemaphoreType.DMA,
            pltpu.SemaphoreType.DMA,
        ],
    )(a_flat, params)


def _scan1_body(kk_ref, hist_ref, bs_ref, meta_ref):
    tot = jnp.sum(hist_ref[...], axis=0)
    kk = kk_ref[0]
    iota = (lax.broadcasted_iota(jnp.int32, (_NBIN // 128, 128), 0) * 128
            + lax.broadcasted_iota(jnp.int32, (_NBIN // 128, 128), 1))

    def suffix(b):
        return jnp.sum(jnp.where(iota >= b, tot, 0))

    def bs_body(_, carry):
        lo_b, hi_b = carry
        mid = (lo_b + hi_b) // 2
        ge = suffix(mid) >= kk
        return jnp.where(ge, mid, lo_b), jnp.where(ge, hi_b, mid)

    bstar, _ = lax.fori_loop(0, 12, bs_body, (jnp.int32(0), jnp.int32(_NBIN)))
    s_next = jnp.sum(jnp.where(iota >= bstar + 1, tot, 0))
    bs_ref[...] = jnp.full((1, _NLANE), bstar, jnp.int32)
    lane = lax.broadcasted_iota(jnp.int32, (1, _NLANE), 1)
    meta_ref[...] = jnp.where(lane == 0, s_next, 0).astype(jnp.int32)


def _scan1(hist, kk):
    return pl.pallas_call(
        _scan1_body,
        in_specs=[
            pl.BlockSpec(memory_space=pltpu.SMEM),
            pl.BlockSpec((2, _NBIN // 128, 128), lambda: (0, 0, 0)),
        ],
        out_specs=[
            pl.BlockSpec((1, _NLANE), lambda: (0, 0)),
            pl.BlockSpec((1, _NLANE), lambda: (0, 0)),
        ],
        out_shape=[
            jax.ShapeDtypeStruct((1, _NLANE), jnp.int32),
            jax.ShapeDtypeStruct((1, _NLANE), jnp.int32),
        ],
        interpret=_INTERPRET,
    )(kk.reshape(1), hist)


def _scan2_body(kk_ref, hist_ref, bs_ref, meta_ref, out_ref):
    tot = jnp.sum(hist_ref[...], axis=0)
    kk = kk_ref[0]
    bstar = bs_ref[0, 0]
    s_next = meta_ref[0, 0]
    iota = (lax.broadcasted_iota(jnp.int32, (_NBIN // 128, 128), 0) * 128
            + lax.broadcasted_iota(jnp.int32, (_NBIN // 128, 128), 1))

    def suffix(c):
        return s_next + jnp.sum(jnp.where(iota >= c, tot, 0))

    def bs_body(_, carry):
        lo_c, hi_c = carry
        mid = (lo_c + hi_c) // 2
        ge = suffix(mid) >= kk
        return jnp.where(ge, mid, lo_c), jnp.where(ge, hi_c, mid)

    cstar, _ = lax.fori_loop(0, 12, bs_body, (jnp.int32(0), jnp.int32(_NBIN)))
    clo = suffix(cstar)
    chi = suffix(cstar + 1)
    base = bstar * 524288
    lo2 = base + cstar * 128
    lane = lax.broadcasted_iota(jnp.int32, (1, _NLANE), 1)
    vals = (jnp.where(lane == 0, lo2, 0) + jnp.where(lane == 1, lo2 + 128, 0)
            + jnp.where(lane == 2, clo, 0) + jnp.where(lane == 3, chi, 0))
    out_ref[...] = vals.astype(jnp.int32)


def _scan2(hist2, bs, meta, kk):
    return pl.pallas_call(
        _scan2_body,
        in_specs=[
            pl.BlockSpec(memory_space=pltpu.SMEM),
            pl.BlockSpec((2, _NBIN // 128, 128), lambda: (0, 0, 0)),
            pl.BlockSpec(memory_space=pltpu.SMEM),
            pl.BlockSpec(memory_space=pltpu.SMEM),
        ],
        out_specs=pl.BlockSpec((1, _NLANE), lambda: (0, 0)),
        out_shape=jax.ShapeDtypeStruct((1, _NLANE), jnp.int32),
        interpret=_INTERPRET,
    )(kk.reshape(1), hist2, bs, meta)


def _decode_body(tb_ref, a_ref, wd_ref, bd_ref, xhat_ref, z_ref, nnz_ref,
                 sz_ref):
    j = pl.program_id(0)
    a = a_ref[...]
    bits = jax.lax.bitcast_convert_type(a, jnp.int32)
    z = jnp.where(bits >= tb_ref[0], a, 0.0)
    z_ref[...] = z
    part = jax.lax.dot_general(
        z, wd_ref[...], (((1,), (1,)), ((), ())),
        preferred_element_type=jnp.float32)
    nz = jnp.sum((z > 0.0).astype(jnp.int32))
    sz = jnp.sum(z)

    @pl.when(j == 0)
    def _():
        xhat_ref[...] = bd_ref[...] + part
        nnz_ref[0] = nz
        sz_ref[0] = sz

    @pl.when(j > 0)
    def _():
        xhat_ref[...] += part
        nnz_ref[0] += nz
        sz_ref[0] += sz


def _decode(a, t_bits, W_dec_w, W_dec_b):
    nsteps = 32
    bn = D_SAE // nsteps
    return pl.pallas_call(
        _decode_body,
        grid=(nsteps,),
        in_specs=[
            pl.BlockSpec(memory_space=pltpu.SMEM),
            pl.BlockSpec((B, bn), lambda j: (0, j)),
            pl.BlockSpec((D_IN, bn), lambda j: (0, j)),
            pl.BlockSpec((1, D_IN), lambda j: (0, 0)),
        ],
        out_specs=[
            pl.BlockSpec((B, D_IN), lambda j: (0, 0)),
            pl.BlockSpec((B, bn), lambda j: (0, j)),
            pl.BlockSpec(memory_space=pltpu.SMEM),
            pl.BlockSpec(memory_space=pltpu.SMEM),
        ],
        out_shape=[
            jax.ShapeDtypeStruct((B, D_IN), jnp.float32),
            jax.ShapeDtypeStruct((B, D_SAE), jnp.float32),
            jax.ShapeDtypeStruct((1,), jnp.int32),
            jax.ShapeDtypeStruct((1,), jnp.float32),
        ],
        interpret=_INTERPRET,
    )(t_bits, a, W_dec_w, W_dec_b.reshape(1, D_IN))


def kernel(x, W_enc_w, W_enc_b, W_dec_w, W_dec_b, k_total):
    a, mx = _encode(x, W_enc_w, W_enc_b)
    kk = jnp.clip(jnp.asarray(k_total, jnp.int32), 1, K_MAX)
    mx_bits = jax.lax.bitcast_convert_type(mx[0, 0], jnp.int32)

    def make_body(rows, nsteps, target, first_pstar, first_w):
        def body(carry):
            lo, hi, clo, chi, it = carry
            width = hi - lo
            frac = (clo - target).astype(jnp.float32) / jnp.maximum(
                (clo - chi).astype(jnp.float32), 1.0)
            pstar = lo + (frac * width.astype(jnp.float32)).astype(jnp.int32)
            w = jnp.maximum(width // 1024, 1)
            if first_pstar is not None:
                pstar = jnp.where(it == 0, first_pstar, pstar)
                w = jnp.where(it == 0, first_w, w)
            offs = jnp.array([-64, -16, -4, 0, 4, 16, 64], dtype=jnp.int32)
            interp_pts = jnp.concatenate([
                pstar + offs * w,
                jnp.stack([lo + width // 2, lo + 1]),
            ])
            sweep_pts = lo + jnp.arange(1, NTH + 1, dtype=jnp.int32)
            pts = jnp.where(width <= NTH + 1, sweep_pts, interp_pts)
            pts = jnp.sort(jnp.clip(pts, lo + 1, hi))
            cnts = _count_pass(a, pts, rows, nsteps)
            ge = cnts >= target
            new_lo = jnp.max(jnp.where(ge, pts, lo))
            new_hi = jnp.min(jnp.where(ge, hi, pts))
            new_clo = jnp.min(jnp.where(ge, cnts, clo))
            new_chi = jnp.max(jnp.where(ge, chi, cnts))
            return new_lo, new_hi, new_clo, new_chi, it + 1

        return body

    del mx_bits

    a_flat = jax.lax.bitcast_convert_type(a, jnp.int32).reshape(-1)
    hist1 = _sc_hist(a_flat, jnp.zeros((1, _NLANE), jnp.int32), level=1)
    bs, meta = _scan1(hist1, kk)
    hist2 = _sc_hist(a_flat, bs, level=2)
    out2 = _scan2(hist2, bs, meta, kk)
    lo2, hi2 = out2[0, 0], out2[0, 1]
    clo2, chi2 = out2[0, 2], out2[0, 3]

    def full_cond(carry):
        lo, hi, clo, chi, _ = carry
        return jnp.logical_and(hi - lo > 1, clo - chi > 4)

    lo, _, _, _, _ = jax.lax.while_loop(
        full_cond, make_body(B, 16, kk, None, None),
        (lo2, hi2, clo2, chi2, jnp.int32(1)))

    x_hat, z, nnz, sz = _decode(a, lo.reshape(1), W_dec_w, W_dec_b)
    nnz_s = nnz[0]
    frac_nnz = nnz_s.astype(jnp.float32) / jnp.float32(B * D_SAE)
    mean_active = sz[0] / jnp.maximum(nnz_s.astype(jnp.float32), 1.0)
    return (x_hat, z, frac_nnz, mean_active, nnz_s)


# speedup vs baseline: 1.2037x; 1.2037x over previous
import functools

import jax
import jax.numpy as jnp
from jax import lax
from jax.experimental import pallas as pl
from jax.experimental.pallas import tpu as pltpu
from jax.experimental.pallas import tpu_sc as plsc

B = 2048
D_IN = 1024
D_SAE = 16384
K_MAX = 131072
NTH = 9

_INTERPRET = False


def _encode_body(x_ref, w_ref, b_ref, a_ref, mx_ref):
    j = pl.program_id(0)
    acc = jax.lax.dot_general(
        x_ref[...], w_ref[...], (((1,), (1,)), ((), ())),
        preferred_element_type=jnp.float32)
    a = jnp.maximum(acc + b_ref[...], 0.0)
    a_ref[...] = jax.lax.bitcast_convert_type(a, jnp.int32)
    m = jnp.max(a)

    @pl.when(j == 0)
    def _():
        mx_ref[...] = jnp.full((1, 1), m, jnp.float32)

    @pl.when(j > 0)
    def _():
        mx_ref[...] = jnp.maximum(mx_ref[...], jnp.full((1, 1), m, jnp.float32))


def _encode(x, W_enc_w, W_enc_b):
    nsteps = 16
    bn = D_SAE // nsteps
    return pl.pallas_call(
        _encode_body,
        grid=(nsteps,),
        in_specs=[
            pl.BlockSpec((B, D_IN), lambda j: (0, 0)),
            pl.BlockSpec((bn, D_IN), lambda j: (j, 0)),
            pl.BlockSpec((1, bn), lambda j: (0, j)),
        ],
        out_specs=[
            pl.BlockSpec((B, bn), lambda j: (0, j)),
            pl.BlockSpec((1, 1), lambda j: (0, 0)),
        ],
        out_shape=[
            jax.ShapeDtypeStruct((B, D_SAE), jnp.int32),
            jax.ShapeDtypeStruct((1, 1), jnp.float32),
        ],
        interpret=_INTERPRET,
    )(x, W_enc_w, W_enc_b.reshape(1, D_SAE))


def _count_body(thr_ref, a_ref, cnt_ref, *, nsteps):
    j = pl.program_id(0)
    bits = a_ref[...]

    @pl.when(j == 0)
    def _():
        for i in range(NTH):
            cnt_ref[i] = 0

    for i in range(NTH):
        cnt_ref[i] += jnp.sum((bits >= thr_ref[i]).astype(jnp.int32))


def _count_pass(a, thr_bits, rows, nsteps):
    bm = rows // nsteps
    return pl.pallas_call(
        functools.partial(_count_body, nsteps=nsteps),
        grid=(nsteps,),
        in_specs=[
            pl.BlockSpec(memory_space=pltpu.SMEM),
            pl.BlockSpec((bm, D_SAE), lambda j: (j, 0)),
        ],
        out_specs=pl.BlockSpec(memory_space=pltpu.SMEM),
        out_shape=jax.ShapeDtypeStruct((NTH,), jnp.int32),
        interpret=_INTERPRET,
    )(thr_bits, a)


_NLANE = 16
_NBIN = 4096
_CH = 8192
_NELEM = B * D_SAE


def _sc_hist_body(level, a_ref, par_ref, out_ref, buf0, buf1, hist, fold,
                  shared, pv, idxv, sem0, sem1):
    cid = lax.axis_index("c")
    sid = lax.axis_index("s")
    wid = sid * 2 + cid
    nw = 32
    slab = _NELEM // nw
    nch = slab // _CH
    base = wid * slab

    l16 = jnp.arange(_NLANE, dtype=jnp.int32)
    ones = jnp.ones((_NLANE,), jnp.int32)
    zeros16 = jnp.zeros((_NLANE,), jnp.int32)

    def zhist(i, c):
        hist[pl.ds(i * _NLANE, _NLANE)] = zeros16
        return c

    lax.fori_loop(0, (_NBIN * _NLANE) // _NLANE, zhist, 0)

    def zfold(i, c):
        fold[i // 8, pl.ds((i % 8) * _NLANE, _NLANE)] = zeros16
        return c

    lax.fori_loop(0, _NBIN // _NLANE, zfold, 0)

    idxv[pl.ds(0, _NLANE)] = l16
    idxv[pl.ds(_NLANE, _NLANE)] = l16 + _NLANE

    @pl.when(sid == 0)
    def _():
        pltpu.sync_copy(fold, shared)

    if level == 2:
        pltpu.sync_copy(par_ref.at[0], pv)
    plsc.subcore_barrier()

    def process(buf):
        bsel = pv[...] if level == 2 else None

        def ib(i, c):
            for u in range(8):
                bits = buf[pl.ds(i * 128 + u * 16, 16)]
                top = lax.shift_right_logical(bits, 19)
                if level == 1:
                    idx = top * _NLANE + l16
                    plsc.addupdate_scatter(hist, [idx], ones)
                else:
                    mid = jnp.bitwise_and(
                        lax.shift_right_logical(bits, 7), _NBIN - 1)
                    idx = mid * _NLANE + l16
                    plsc.addupdate_scatter(hist, [idx], ones,
                                           mask=top == bsel)
            return c

        lax.fori_loop(0, _CH // 128, ib, 0)

    pltpu.async_copy(a_ref.at[pl.ds(base, _CH)], buf0, sem0)

    def pair(p, c):
        pltpu.async_copy(
            a_ref.at[pl.ds(base + (2 * p + 1) * _CH, _CH)], buf1, sem1)
        pltpu.make_async_copy(a_ref.at[pl.ds(0, _CH)], buf0, sem0).wait()
        process(buf0)

        @pl.when(2 * p + 2 < nch)
        def _():
            pltpu.async_copy(
                a_ref.at[pl.ds(base + (2 * p + 2) * _CH, _CH)], buf0, sem0)

        pltpu.make_async_copy(a_ref.at[pl.ds(0, _CH)], buf1, sem1).wait()
        process(buf1)
        return c

    lax.fori_loop(0, nch // 2, pair, 0)

    def fld(cb, c):
        acc = zeros16
        for j in range(_NLANE):
            s = jnp.sum(hist[pl.ds((cb * _NLANE + j) * _NLANE, _NLANE)])
            acc = acc + jnp.where(l16 == j, s, 0)
        fold[cb // 8, pl.ds((cb % 8) * _NLANE, _NLANE)] = acc
        return c

    lax.fori_loop(0, _NBIN // _NLANE, fld, 0)

    pltpu.sync_copy(fold, shared.at[idxv], add=True)
    plsc.subcore_barrier()

    @pl.when(sid == 0)
    def _():
        pltpu.sync_copy(shared, out_ref.at[cid])


def _sc_hist(a_flat, params, level):
    mesh = plsc.VectorSubcoreMesh(core_axis_name="c", subcore_axis_name="s")
    return pl.kernel(
        functools.partial(_sc_hist_body, level),
        mesh=mesh,
        compiler_params=pltpu.CompilerParams(needs_layout_passes=False),
        out_type=jax.ShapeDtypeStruct((2, _NBIN // 128, 128), jnp.int32),
        scratch_types=[
            pltpu.VMEM((_CH,), jnp.int32),
            pltpu.VMEM((_CH,), jnp.int32),
            pltpu.VMEM((_NBIN * _NLANE,), jnp.int32),
            pltpu.VMEM((_NBIN // 128, 128), jnp.int32),
            pltpu.VMEM_SHARED((_NBIN // 128, 128), jnp.int32),
            pltpu.VMEM((_NLANE,), jnp.int32),
            pltpu.VMEM((_NBIN // 128,), jnp.int32),
            pltpu.SemaphoreType.DMA,
            pltpu.SemaphoreType.DMA,
        ],
    )(a_flat, params)


def _scan1_body(kk_ref, hist_ref, bs_ref, meta_ref):
    tot = jnp.sum(hist_ref[...], axis=0)
    kk = kk_ref[0]
    iota = (lax.broadcasted_iota(jnp.int32, (_NBIN // 128, 128), 0) * 128
            + lax.broadcasted_iota(jnp.int32, (_NBIN // 128, 128), 1))

    def suffix(b):
        return jnp.sum(jnp.where(iota >= b, tot, 0))

    def bs_body(_, carry):
        lo_b, hi_b = carry
        mid = (lo_b + hi_b) // 2
        ge = suffix(mid) >= kk
        return jnp.where(ge, mid, lo_b), jnp.where(ge, hi_b, mid)

    bstar, _ = lax.fori_loop(0, 12, bs_body, (jnp.int32(0), jnp.int32(_NBIN)))
    s_next = jnp.sum(jnp.where(iota >= bstar + 1, tot, 0))
    bs_ref[...] = jnp.full((1, _NLANE), bstar, jnp.int32)
    lane = lax.broadcasted_iota(jnp.int32, (1, _NLANE), 1)
    meta_ref[...] = jnp.where(lane == 0, s_next, 0).astype(jnp.int32)


def _scan1(hist, kk):
    return pl.pallas_call(
        _scan1_body,
        in_specs=[
            pl.BlockSpec(memory_space=pltpu.SMEM),
            pl.BlockSpec((2, _NBIN // 128, 128), lambda: (0, 0, 0)),
        ],
        out_specs=[
            pl.BlockSpec((1, _NLANE), lambda: (0, 0)),
            pl.BlockSpec((1, _NLANE), lambda: (0, 0)),
        ],
        out_shape=[
            jax.ShapeDtypeStruct((1, _NLANE), jnp.int32),
            jax.ShapeDtypeStruct((1, _NLANE), jnp.int32),
        ],
        interpret=_INTERPRET,
    )(kk.reshape(1), hist)


def _scan2_body(kk_ref, hist_ref, bs_ref, meta_ref, out_ref):
    tot = jnp.sum(hist_ref[...], axis=0)
    kk = kk_ref[0]
    bstar = bs_ref[0, 0]
    s_next = meta_ref[0, 0]
    iota = (lax.broadcasted_iota(jnp.int32, (_NBIN // 128, 128), 0) * 128
            + lax.broadcasted_iota(jnp.int32, (_NBIN // 128, 128), 1))

    def suffix(c):
        return s_next + jnp.sum(jnp.where(iota >= c, tot, 0))

    def bs_body(_, carry):
        lo_c, hi_c = carry
        mid = (lo_c + hi_c) // 2
        ge = suffix(mid) >= kk
        return jnp.where(ge, mid, lo_c), jnp.where(ge, hi_c, mid)

    cstar, _ = lax.fori_loop(0, 12, bs_body, (jnp.int32(0), jnp.int32(_NBIN)))
    clo = suffix(cstar)
    chi = suffix(cstar + 1)
    base = bstar * 524288
    lo2 = base + cstar * 128
    lane = lax.broadcasted_iota(jnp.int32, (1, _NLANE), 1)
    vals = (jnp.where(lane == 0, lo2, 0) + jnp.where(lane == 1, lo2 + 128, 0)
            + jnp.where(lane == 2, clo, 0) + jnp.where(lane == 3, chi, 0))
    out_ref[...] = vals.astype(jnp.int32)


def _scan2(hist2, bs, meta, kk):
    return pl.pallas_call(
        _scan2_body,
        in_specs=[
            pl.BlockSpec(memory_space=pltpu.SMEM),
            pl.BlockSpec((2, _NBIN // 128, 128), lambda: (0, 0, 0)),
            pl.BlockSpec(memory_space=pltpu.SMEM),
            pl.BlockSpec(memory_space=pltpu.SMEM),
        ],
        out_specs=pl.BlockSpec((1, _NLANE), lambda: (0, 0)),
        out_shape=jax.ShapeDtypeStruct((1, _NLANE), jnp.int32),
        interpret=_INTERPRET,
    )(kk.reshape(1), hist2, bs, meta)


def _decode_body(tb_ref, a_ref, wd_ref, bd_ref, xhat_ref, z_ref, nnz_ref,
                 sz_ref):
    j = pl.program_id(0)
    bits = a_ref[...]
    a = jax.lax.bitcast_convert_type(bits, jnp.float32)
    z = jnp.where(bits >= tb_ref[0], a, 0.0)
    z_ref[...] = z
    part = jax.lax.dot_general(
        z, wd_ref[...], (((1,), (1,)), ((), ())),
        preferred_element_type=jnp.float32)
    nz = jnp.sum((z > 0.0).astype(jnp.int32))
    sz = jnp.sum(z)

    @pl.when(j == 0)
    def _():
        xhat_ref[...] = bd_ref[...] + part
        nnz_ref[0] = nz
        sz_ref[0] = sz

    @pl.when(j > 0)
    def _():
        xhat_ref[...] += part
        nnz_ref[0] += nz
        sz_ref[0] += sz


def _decode(a, t_bits, W_dec_w, W_dec_b):
    nsteps = 32
    bn = D_SAE // nsteps
    return pl.pallas_call(
        _decode_body,
        grid=(nsteps,),
        in_specs=[
            pl.BlockSpec(memory_space=pltpu.SMEM),
            pl.BlockSpec((B, bn), lambda j: (0, j)),
            pl.BlockSpec((D_IN, bn), lambda j: (0, j)),
            pl.BlockSpec((1, D_IN), lambda j: (0, 0)),
        ],
        out_specs=[
            pl.BlockSpec((B, D_IN), lambda j: (0, 0)),
            pl.BlockSpec((B, bn), lambda j: (0, j)),
            pl.BlockSpec(memory_space=pltpu.SMEM),
            pl.BlockSpec(memory_space=pltpu.SMEM),
        ],
        out_shape=[
            jax.ShapeDtypeStruct((B, D_IN), jnp.float32),
            jax.ShapeDtypeStruct((B, D_SAE), jnp.float32),
            jax.ShapeDtypeStruct((1,), jnp.int32),
            jax.ShapeDtypeStruct((1,), jnp.float32),
        ],
        interpret=_INTERPRET,
    )(t_bits, a, W_dec_w, W_dec_b.reshape(1, D_IN))


def kernel(x, W_enc_w, W_enc_b, W_dec_w, W_dec_b, k_total):
    a, mx = _encode(x, W_enc_w, W_enc_b)
    kk = jnp.clip(jnp.asarray(k_total, jnp.int32), 1, K_MAX)
    mx_bits = jax.lax.bitcast_convert_type(mx[0, 0], jnp.int32)

    def make_body(rows, nsteps, target, first_pstar, first_w):
        def body(carry):
            lo, hi, clo, chi, it = carry
            width = hi - lo
            frac = (clo - target).astype(jnp.float32) / jnp.maximum(
                (clo - chi).astype(jnp.float32), 1.0)
            pstar = lo + (frac * width.astype(jnp.float32)).astype(jnp.int32)
            w = jnp.maximum(width // 1024, 1)
            if first_pstar is not None:
                pstar = jnp.where(it == 0, first_pstar, pstar)
                w = jnp.where(it == 0, first_w, w)
            offs = jnp.array([-64, -16, -4, 0, 4, 16, 64], dtype=jnp.int32)
            interp_pts = jnp.concatenate([
                pstar + offs * w,
                jnp.stack([lo + width // 2, lo + 1]),
            ])
            sweep_pts = lo + jnp.arange(1, NTH + 1, dtype=jnp.int32)
            pts = jnp.where(width <= NTH + 1, sweep_pts, interp_pts)
            pts = jnp.sort(jnp.clip(pts, lo + 1, hi))
            cnts = _count_pass(a, pts, rows, nsteps)
            ge = cnts >= target
            new_lo = jnp.max(jnp.where(ge, pts, lo))
            new_hi = jnp.min(jnp.where(ge, hi, pts))
            new_clo = jnp.min(jnp.where(ge, cnts, clo))
            new_chi = jnp.max(jnp.where(ge, chi, cnts))
            return new_lo, new_hi, new_clo, new_chi, it + 1

        return body

    del mx_bits

    a_flat = a.reshape(-1)
    hist1 = _sc_hist(a_flat, jnp.zeros((1, _NLANE), jnp.int32), level=1)
    bs, meta = _scan1(hist1, kk)
    hist2 = _sc_hist(a_flat, bs, level=2)
    out2 = _scan2(hist2, bs, meta, kk)
    lo2, hi2 = out2[0, 0], out2[0, 1]
    clo2, chi2 = out2[0, 2], out2[0, 3]

    def full_cond(carry):
        lo, hi, clo, chi, _ = carry
        return jnp.logical_and(hi - lo > 1, clo - chi > 4)

    lo, _, _, _, _ = jax.lax.while_loop(
        full_cond, make_body(B, 16, kk, None, None),
        (lo2, hi2, clo2, chi2, jnp.int32(1)))

    x_hat, z, nnz, sz = _decode(a, lo.reshape(1), W_dec_w, W_dec_b)
    nnz_s = nnz[0]
    frac_nnz = nnz_s.astype(jnp.float32) / jnp.float32(B * D_SAE)
    mean_active = sz[0] / jnp.maximum(nnz_s.astype(jnp.float32), 1.0)
    return (x_hat, z, frac_nnz, mean_active, nnz_s)


# speedup vs baseline: 1.7005x; 1.4128x over previous
import functools

import jax
import jax.numpy as jnp
from jax import lax
from jax.experimental import pallas as pl
from jax.experimental.pallas import tpu as pltpu

B = 2048
D_IN = 1024
D_SAE = 16384
K_MAX = 131072
NTH = 9

_INTERPRET = False


def _encode_body(x_ref, w_ref, b_ref, a_ref, mx_ref):
    j = pl.program_id(0)
    acc = jax.lax.dot_general(
        x_ref[...], w_ref[...], (((1,), (1,)), ((), ())),
        preferred_element_type=jnp.float32)
    a = jnp.maximum(acc + b_ref[...], 0.0)
    a_ref[...] = jax.lax.bitcast_convert_type(a, jnp.int32)
    m = jnp.max(a)

    @pl.when(j == 0)
    def _():
        mx_ref[...] = jnp.full((1, 1), m, jnp.float32)

    @pl.when(j > 0)
    def _():
        mx_ref[...] = jnp.maximum(mx_ref[...], jnp.full((1, 1), m, jnp.float32))


def _encode(x, W_enc_w, W_enc_b):
    nsteps = 16
    bn = D_SAE // nsteps
    return pl.pallas_call(
        _encode_body,
        grid=(nsteps,),
        in_specs=[
            pl.BlockSpec((B, D_IN), lambda j: (0, 0)),
            pl.BlockSpec((bn, D_IN), lambda j: (j, 0)),
            pl.BlockSpec((1, bn), lambda j: (0, j)),
        ],
        out_specs=[
            pl.BlockSpec((B, bn), lambda j: (0, j)),
            pl.BlockSpec((1, 1), lambda j: (0, 0)),
        ],
        out_shape=[
            jax.ShapeDtypeStruct((B, D_SAE), jnp.int32),
            jax.ShapeDtypeStruct((1, 1), jnp.float32),
        ],
        interpret=_INTERPRET,
    )(x, W_enc_w, W_enc_b.reshape(1, D_SAE))


def _count_body(thr_ref, a_ref, cnt_ref, *, nsteps):
    j = pl.program_id(0)
    bits = a_ref[...]

    @pl.when(j == 0)
    def _():
        for i in range(NTH):
            cnt_ref[i] = 0

    for i in range(NTH):
        cnt_ref[i] += jnp.sum((bits >= thr_ref[i]).astype(jnp.int32))


def _count_pass(a, thr_bits, rows, nsteps):
    bm = rows // nsteps
    return pl.pallas_call(
        functools.partial(_count_body, nsteps=nsteps),
        grid=(nsteps,),
        in_specs=[
            pl.BlockSpec(memory_space=pltpu.SMEM),
            pl.BlockSpec((bm, D_SAE), lambda j: (j, 0)),
        ],
        out_specs=pl.BlockSpec(memory_space=pltpu.SMEM),
        out_shape=jax.ShapeDtypeStruct((NTH,), jnp.int32),
        interpret=_INTERPRET,
    )(thr_bits, a)


_SUB_BLOCKS = 2
_SUB_PASSES = 3
_FULL_PASSES = 3
_BLK_ROWS = 128
_NBLK = B // _BLK_ROWS
_SUB_STEPS = _SUB_PASSES * _SUB_BLOCKS
_NSTEP = _SUB_STEPS + _FULL_PASSES * _NBLK
_LADDER = tuple((127 + e) << 23 for e in (-12, -8, -4, -2, 0, 1, 2, 4, 8))
_OFFS = (-64, -16, -4, 0, 4, 16, 64)
_HI0 = 0x7F800000


def _search_body(kk_ref, a_ref, out_ref, st_ref, thr_ref, cnt_ref):
    s = pl.program_id(0)
    in_sub = s < _SUB_STEPS
    pass_start = jnp.where(in_sub, s % _SUB_BLOCKS == 0,
                           (s - _SUB_STEPS) % _NBLK == 0)
    pass_end = jnp.where(in_sub, s % _SUB_BLOCKS == _SUB_BLOCKS - 1,
                         (s - _SUB_STEPS) % _NBLK == _NBLK - 1)
    p = jnp.where(in_sub, s // _SUB_BLOCKS,
                  _SUB_PASSES + (s - _SUB_STEPS) // _NBLK)

    @pl.when(s == 0)
    def _():
        st_ref[0] = 0
        st_ref[1] = _HI0
        st_ref[2] = _SUB_BLOCKS * _BLK_ROWS * D_SAE
        st_ref[3] = 0
        for i in range(NTH):
            thr_ref[i] = _LADDER[i]

    bits = a_ref[...]
    for i in range(NTH):
        c = jnp.sum((bits >= thr_ref[i]).astype(jnp.int32))
        cnt_ref[i] = jnp.where(pass_start, c, cnt_ref[i] + c)

    @pl.when(pass_end)
    def _():
        kk = kk_ref[0]
        kk_sub = jnp.maximum(kk // (B // (_SUB_BLOCKS * _BLK_ROWS)), 1)
        target = jnp.where(p < _SUB_PASSES, kk_sub, kk)
        lo, hi = st_ref[0], st_ref[1]
        clo, chi = st_ref[2], st_ref[3]
        for i in range(NTH):
            pt = thr_ref[i]
            c = cnt_ref[i]
            ge = c >= target
            take_lo = jnp.logical_and(ge, pt > lo)
            lo = jnp.where(take_lo, pt, lo)
            clo = jnp.where(take_lo, c, clo)
            take_hi = jnp.logical_and(~ge, pt < hi)
            hi = jnp.where(take_hi, pt, hi)
            chi = jnp.where(take_hi, c, chi)

        at_seed = p == _SUB_PASSES - 1
        seed = lo
        lo = jnp.where(at_seed, 0, lo)
        hi = jnp.where(at_seed, _HI0, hi)
        clo = jnp.where(at_seed, B * D_SAE, clo)
        chi = jnp.where(at_seed, 0, chi)

        width = hi - lo
        tnext = jnp.where(p >= _SUB_PASSES - 1, kk, kk_sub)
        frac = (clo - tnext).astype(jnp.float32) / jnp.maximum(
            (clo - chi).astype(jnp.float32), 1.0)
        pstar = lo + (frac * width.astype(jnp.float32)).astype(jnp.int32)
        w = jnp.maximum(width // 1024, 1)
        pstar = jnp.where(at_seed, seed, pstar)
        w = jnp.where(at_seed, 2048, w)
        sweep = width <= NTH
        for i in range(len(_OFFS)):
            pt = jnp.clip(pstar + _OFFS[i] * w, lo + 1, hi)
            thr_ref[i] = jnp.where(sweep, jnp.minimum(lo + 1 + i, hi), pt)
        thr_ref[7] = jnp.where(sweep, jnp.minimum(lo + 8, hi),
                               lo + width // 2)
        thr_ref[8] = lo + 1
        st_ref[0] = lo
        st_ref[1] = hi
        st_ref[2] = clo
        st_ref[3] = chi

    @pl.when(s == _NSTEP - 1)
    def _():
        for i in range(4):
            out_ref[i] = st_ref[i]


def _search(a, kk):
    return pl.pallas_call(
        _search_body,
        grid=(_NSTEP,),
        in_specs=[
            pl.BlockSpec(memory_space=pltpu.SMEM),
            pl.BlockSpec(
                (_BLK_ROWS, D_SAE),
                lambda s: (jnp.where(s < _SUB_STEPS, s % _SUB_BLOCKS,
                                     (s - _SUB_STEPS) % _NBLK), 0)),
        ],
        out_specs=pl.BlockSpec(memory_space=pltpu.SMEM),
        out_shape=jax.ShapeDtypeStruct((4,), jnp.int32),
        scratch_shapes=[
            pltpu.SMEM((4,), jnp.int32),
            pltpu.SMEM((NTH,), jnp.int32),
            pltpu.SMEM((NTH,), jnp.int32),
        ],
        interpret=_INTERPRET,
    )(kk.reshape(1), a)


def _decode_body(tb_ref, a_ref, wd_ref, bd_ref, xhat_ref, z_ref, nnz_ref,
                 sz_ref):
    j = pl.program_id(0)
    bits = a_ref[...]
    a = jax.lax.bitcast_convert_type(bits, jnp.float32)
    z = jnp.where(bits >= tb_ref[0], a, 0.0)
    z_ref[...] = z
    part = jax.lax.dot_general(
        z, wd_ref[...], (((1,), (1,)), ((), ())),
        preferred_element_type=jnp.float32)
    nz = jnp.sum((z > 0.0).astype(jnp.int32))
    sz = jnp.sum(z)

    @pl.when(j == 0)
    def _():
        xhat_ref[...] = bd_ref[...] + part
        nnz_ref[0] = nz
        sz_ref[0] = sz

    @pl.when(j > 0)
    def _():
        xhat_ref[...] += part
        nnz_ref[0] += nz
        sz_ref[0] += sz


def _decode(a, t_bits, W_dec_w, W_dec_b):
    nsteps = 32
    bn = D_SAE // nsteps
    return pl.pallas_call(
        _decode_body,
        grid=(nsteps,),
        in_specs=[
            pl.BlockSpec(memory_space=pltpu.SMEM),
            pl.BlockSpec((B, bn), lambda j: (0, j)),
            pl.BlockSpec((D_IN, bn), lambda j: (0, j)),
            pl.BlockSpec((1, D_IN), lambda j: (0, 0)),
        ],
        out_specs=[
            pl.BlockSpec((B, D_IN), lambda j: (0, 0)),
            pl.BlockSpec((B, bn), lambda j: (0, j)),
            pl.BlockSpec(memory_space=pltpu.SMEM),
            pl.BlockSpec(memory_space=pltpu.SMEM),
        ],
        out_shape=[
            jax.ShapeDtypeStruct((B, D_IN), jnp.float32),
            jax.ShapeDtypeStruct((B, D_SAE), jnp.float32),
            jax.ShapeDtypeStruct((1,), jnp.int32),
            jax.ShapeDtypeStruct((1,), jnp.float32),
        ],
        interpret=_INTERPRET,
    )(t_bits, a, W_dec_w, W_dec_b.reshape(1, D_IN))


def kernel(x, W_enc_w, W_enc_b, W_dec_w, W_dec_b, k_total):
    a, mx = _encode(x, W_enc_w, W_enc_b)
    kk = jnp.clip(jnp.asarray(k_total, jnp.int32), 1, K_MAX)
    mx_bits = jax.lax.bitcast_convert_type(mx[0, 0], jnp.int32)

    def make_body(rows, nsteps, target, first_pstar, first_w):
        def body(carry):
            lo, hi, clo, chi, it = carry
            width = hi - lo
            frac = (clo - target).astype(jnp.float32) / jnp.maximum(
                (clo - chi).astype(jnp.float32), 1.0)
            pstar = lo + (frac * width.astype(jnp.float32)).astype(jnp.int32)
            w = jnp.maximum(width // 1024, 1)
            if first_pstar is not None:
                pstar = jnp.where(it == 0, first_pstar, pstar)
                w = jnp.where(it == 0, first_w, w)
            offs = jnp.array([-64, -16, -4, 0, 4, 16, 64], dtype=jnp.int32)
            interp_pts = jnp.concatenate([
                pstar + offs * w,
                jnp.stack([lo + width // 2, lo + 1]),
            ])
            sweep_pts = lo + jnp.arange(1, NTH + 1, dtype=jnp.int32)
            pts = jnp.where(width <= NTH + 1, sweep_pts, interp_pts)
            pts = jnp.sort(jnp.clip(pts, lo + 1, hi))
            cnts = _count_pass(a, pts, rows, nsteps)
            ge = cnts >= target
            new_lo = jnp.max(jnp.where(ge, pts, lo))
            new_hi = jnp.min(jnp.where(ge, hi, pts))
            new_clo = jnp.min(jnp.where(ge, cnts, clo))
            new_chi = jnp.max(jnp.where(ge, chi, cnts))
            return new_lo, new_hi, new_clo, new_chi, it + 1

        return body

    del mx_bits

    res = _search(a, kk)
    lo2, hi2, clo2, chi2 = res[0], res[1], res[2], res[3]

    def full_cond(carry):
        lo, hi, clo, chi, _ = carry
        return jnp.logical_and(hi - lo > 1, clo - chi > 4)

    lo, _, _, _, _ = jax.lax.while_loop(
        full_cond, make_body(B, 16, kk, None, None),
        (lo2, hi2, clo2, chi2, jnp.int32(1)))

    x_hat, z, nnz, sz = _decode(a, lo.reshape(1), W_dec_w, W_dec_b)
    nnz_s = nnz[0]
    frac_nnz = nnz_s.astype(jnp.float32) / jnp.float32(B * D_SAE)
    mean_active = sz[0] / jnp.maximum(nnz_s.astype(jnp.float32), 1.0)
    return (x_hat, z, frac_nnz, mean_active, nnz_s)


# speedup vs baseline: 2.5757x; 1.5146x over previous
import functools

import jax
import jax.numpy as jnp
from jax import lax
from jax.experimental import pallas as pl
from jax.experimental.pallas import tpu as pltpu

B = 2048
D_IN = 1024
D_SAE = 16384
K_MAX = 131072
NTH = 9

_INTERPRET = False


def _encode_body(x_ref, w_ref, b_ref, a_ref, mx_ref):
    j = pl.program_id(0)
    acc = jax.lax.dot_general(
        x_ref[...], w_ref[...], (((1,), (1,)), ((), ())),
        preferred_element_type=jnp.float32)
    a = jnp.maximum(acc + b_ref[...], 0.0)
    a_ref[...] = jax.lax.bitcast_convert_type(a, jnp.int32)
    m = jnp.max(a)

    @pl.when(j == 0)
    def _():
        mx_ref[...] = jnp.full((1, 1), m, jnp.float32)

    @pl.when(j > 0)
    def _():
        mx_ref[...] = jnp.maximum(mx_ref[...], jnp.full((1, 1), m, jnp.float32))


def _encode(x, W_enc_w, W_enc_b):
    nsteps = 16
    bn = D_SAE // nsteps
    return pl.pallas_call(
        _encode_body,
        grid=(nsteps,),
        in_specs=[
            pl.BlockSpec((B, D_IN), lambda j: (0, 0)),
            pl.BlockSpec((bn, D_IN), lambda j: (j, 0)),
            pl.BlockSpec((1, bn), lambda j: (0, j)),
        ],
        out_specs=[
            pl.BlockSpec((B, bn), lambda j: (0, j)),
            pl.BlockSpec((1, 1), lambda j: (0, 0)),
        ],
        out_shape=[
            jax.ShapeDtypeStruct((B, D_SAE), jnp.int32),
            jax.ShapeDtypeStruct((1, 1), jnp.float32),
        ],
        interpret=_INTERPRET,
    )(x, W_enc_w, W_enc_b.reshape(1, D_SAE))


def _count_body(thr_ref, a_ref, cnt_ref, *, nsteps):
    j = pl.program_id(0)
    bits = a_ref[...]

    @pl.when(j == 0)
    def _():
        for i in range(NTH):
            cnt_ref[i] = 0

    for i in range(NTH):
        cnt_ref[i] += jnp.sum((bits >= thr_ref[i]).astype(jnp.int32))


def _count_pass(a, thr_bits, rows, nsteps):
    bm = rows // nsteps
    return pl.pallas_call(
        functools.partial(_count_body, nsteps=nsteps),
        grid=(nsteps,),
        in_specs=[
            pl.BlockSpec(memory_space=pltpu.SMEM),
            pl.BlockSpec((bm, D_SAE), lambda j: (j, 0)),
        ],
        out_specs=pl.BlockSpec(memory_space=pltpu.SMEM),
        out_shape=jax.ShapeDtypeStruct((NTH,), jnp.int32),
        interpret=_INTERPRET,
    )(thr_bits, a)


def _decode_body(tb_ref, a_ref, wd_ref, bd_ref, xhat_ref, z_ref, nnz_ref,
                 sz_ref):
    j = pl.program_id(0)
    bits = a_ref[...]
    a = jax.lax.bitcast_convert_type(bits, jnp.float32)
    z = jnp.where(bits >= tb_ref[0], a, 0.0)
    z_ref[...] = z
    part = jax.lax.dot_general(
        z, wd_ref[...], (((1,), (1,)), ((), ())),
        preferred_element_type=jnp.float32)
    nz = jnp.sum((z > 0.0).astype(jnp.int32))
    sz = jnp.sum(z)

    @pl.when(j == 0)
    def _():
        xhat_ref[...] = bd_ref[...] + part
        nnz_ref[0] = nz
        sz_ref[0] = sz

    @pl.when(j > 0)
    def _():
        xhat_ref[...] += part
        nnz_ref[0] += nz
        sz_ref[0] += sz


def _decode(a, t_bits, W_dec_w, W_dec_b):
    nsteps = 32
    bn = D_SAE // nsteps
    return pl.pallas_call(
        _decode_body,
        grid=(nsteps,),
        in_specs=[
            pl.BlockSpec(memory_space=pltpu.SMEM),
            pl.BlockSpec((B, bn), lambda j: (0, j)),
            pl.BlockSpec((D_IN, bn), lambda j: (0, j)),
            pl.BlockSpec((1, D_IN), lambda j: (0, 0)),
        ],
        out_specs=[
            pl.BlockSpec((B, D_IN), lambda j: (0, 0)),
            pl.BlockSpec((B, bn), lambda j: (0, j)),
            pl.BlockSpec(memory_space=pltpu.SMEM),
            pl.BlockSpec(memory_space=pltpu.SMEM),
        ],
        out_shape=[
            jax.ShapeDtypeStruct((B, D_IN), jnp.float32),
            jax.ShapeDtypeStruct((B, D_SAE), jnp.float32),
            jax.ShapeDtypeStruct((1,), jnp.int32),
            jax.ShapeDtypeStruct((1,), jnp.float32),
        ],
        interpret=_INTERPRET,
    )(t_bits, a, W_dec_w, W_dec_b.reshape(1, D_IN))


def kernel(x, W_enc_w, W_enc_b, W_dec_w, W_dec_b, k_total):
    a, mx = _encode(x, W_enc_w, W_enc_b)
    kk = jnp.clip(jnp.asarray(k_total, jnp.int32), 1, K_MAX)
    mx_bits = jax.lax.bitcast_convert_type(mx[0, 0], jnp.int32)

    def make_body(rows, nsteps, target, first_pstar, first_w):
        def body(carry):
            lo, hi, clo, chi, it = carry
            width = hi - lo
            frac = (clo - target).astype(jnp.float32) / jnp.maximum(
                (clo - chi).astype(jnp.float32), 1.0)
            pstar = lo + (frac * width.astype(jnp.float32)).astype(jnp.int32)
            w = jnp.maximum(width // 1024, 1)
            if first_pstar is not None:
                pstar = jnp.where(it == 0, first_pstar, pstar)
                w = jnp.where(it == 0, first_w, w)
            offs = jnp.array([-64, -16, -4, 0, 4, 16, 64], dtype=jnp.int32)
            interp_pts = jnp.concatenate([
                pstar + offs * w,
                jnp.stack([lo + width // 2, lo + 1]),
            ])
            sweep_pts = lo + jnp.arange(1, NTH + 1, dtype=jnp.int32)
            pts = jnp.where(width <= NTH + 1, sweep_pts, interp_pts)
            pts = jnp.sort(jnp.clip(pts, lo + 1, hi))
            cnts = _count_pass(a, pts, rows, nsteps)
            ge = cnts >= target
            new_lo = jnp.max(jnp.where(ge, pts, lo))
            new_hi = jnp.min(jnp.where(ge, hi, pts))
            new_clo = jnp.min(jnp.where(ge, cnts, clo))
            new_chi = jnp.max(jnp.where(ge, chi, cnts))
            return new_lo, new_hi, new_clo, new_chi, it + 1

        return body

    hi0 = jnp.maximum(mx_bits, 0) + 1

    sub_rows = 256
    kk_sub = jnp.maximum(kk // (B // sub_rows), 1)

    def sub_cond(carry):
        lo, hi, _, _, it = carry
        return jnp.logical_and(hi - lo > 4096, it < 6)

    sub_lo, _, _, _, _ = jax.lax.while_loop(
        sub_cond, make_body(sub_rows, 2, kk_sub, None, None),
        (jnp.int32(0), hi0, jnp.int32(sub_rows * D_SAE), jnp.int32(0),
         jnp.int32(0)))

    lo2, hi2 = jnp.int32(0), hi0
    clo2, chi2 = jnp.int32(B * D_SAE), jnp.int32(0)

    def full_cond(carry):
        lo, hi, clo, chi, _ = carry
        return jnp.logical_and(hi - lo > 1, clo - chi > 4)

    lo, _, _, _, _ = jax.lax.while_loop(
        full_cond, make_body(B, 16, kk, sub_lo, jnp.int32(2048)),
        (lo2, hi2, clo2, chi2, jnp.int32(0)))

    x_hat, z, nnz, sz = _decode(a, lo.reshape(1), W_dec_w, W_dec_b)
    nnz_s = nnz[0]
    frac_nnz = nnz_s.astype(jnp.float32) / jnp.float32(B * D_SAE)
    mean_active = sz[0] / jnp.maximum(nnz_s.astype(jnp.float32), 1.0)
    return (x_hat, z, frac_nnz, mean_active, nnz_s)
